# baseline (speedup 1.0000x reference)
import jax
import jax.numpy as jnp
import numpy as np
from jax import lax
from jax.experimental import pallas as pl
from jax.experimental.pallas import tpu as pltpu
from jax.experimental.pallas import tpu_sc as plsc

L = 16
NC = 2
NS = 16
NW = NC * NS
CH = 128

MAGIC = np.int32(0x5F3759DF)


def _rsqrt16(d):
    i = lax.bitcast_convert_type(d, jnp.int32)
    y = lax.bitcast_convert_type(MAGIC - (i >> 1), jnp.float32)
    hd = d * 0.5
    y = y * (1.5 - hd * y * y)
    y = y * (1.5 - hd * y * y)
    y = y * (1.5 - hd * y * y)
    return jnp.where(d > 0.0, y, 0.0)


def _make_sc_kernel(npad, d_out, nblk, bs):
    rpt = npad // NS
    qn = d_out // L

    def body(row3d, col3d, ew3d, h_hbm, out_hbm, h2_hbm, dis_hbm,
             idx_r, idx_c, ewb, rows_a, gba, gbb, zbuf, degbuf,
             acc_sh, deg_sh, sga, sgb):
        c = lax.axis_index("c")
        s = lax.axis_index("s")
        wid = s * NC + c
        base_row = s * rpt

        zeros16 = jnp.zeros((L,), jnp.float32)

        @pl.loop(0, 8)
        def _z(i):
            for q in range(qn):
                zbuf[i, pl.ds(q * L, L)] = zeros16

        @pl.loop(0, rpt // 8)
        def _za(k):
            pltpu.sync_copy(zbuf, acc_sh.at[pl.ds(base_row + k * 8, 8)])

        @pl.loop(0, rpt // CH)
        def _zd(k):
            pltpu.sync_copy(zbuf.at[0], deg_sh.at[pl.ds(base_row + k * CH, CH)])

        plsc.subcore_barrier()

        for p in range(2):
            for b in range(nblk):
                pltpu.sync_copy(col3d.at[s * 2 + p, b], idx_c)
                pltpu.sync_copy(ew3d.at[s * 2 + p, b], ewb)

                @pl.loop(0, bs)
                def _fire(j):
                    pltpu.async_copy(ewb.at[j], deg_sh.at[idx_c.at[j]], sga,
                                     add=True)

                @pl.loop(0, bs)
                def _drain(j):
                    pltpu.make_async_copy(ewb.at[j], deg_sh.at[idx_c.at[j]],
                                          sga).wait()

        plsc.subcore_barrier()

        pltpu.sync_copy(deg_sh.at[pl.ds(base_row, rpt)], degbuf)

        @pl.loop(0, rpt // L)
        def _rs(k):
            koff = pl.multiple_of(k * L, L)
            degbuf[pl.ds(koff, L)] = _rsqrt16(degbuf[pl.ds(koff, L)])

        pltpu.sync_copy(degbuf, dis_hbm.at[c, pl.ds(base_row, rpt)])

        @pl.loop(0, rpt // CH)
        def _h2(k):
            roff = pl.multiple_of(k * CH, CH)
            pltpu.sync_copy(h_hbm.at[pl.ds(base_row + roff, CH)], rows_a)

            @pl.loop(0, CH // L)
            def _hg(g):
                goff = pl.multiple_of(g * L, L)
                dv = degbuf[pl.ds(roff + goff, L)]
                for i in range(L):
                    w = dv[i]
                    r_idx = goff + i
                    for q in range(qn):
                        rows_a[r_idx, pl.ds(q * L, L)] = (
                            rows_a[r_idx, pl.ds(q * L, L)] * w)

            @pl.loop(0, CH)
            def _pk(r):
                for q4 in range(qn // 2):
                    a = rows_a[r, pl.ds(q4 * 2 * L, L)]
                    bvec = rows_a[r, pl.ds((q4 * 2 + 1) * L, L)]
                    wa = lax.bitcast_convert_type(a, jnp.int32)
                    wb = lax.bitcast_convert_type(bvec, jnp.int32)
                    ra = (wa + 0x7FFF + ((wa >> 16) & 1)) >> 16
                    rb = (wb + 0x7FFF + ((wb >> 16) & 1)) >> 16
                    gba[r, pl.ds(q4 * L, L)] = (rb << 16) | (ra & 0xFFFF)

            pltpu.sync_copy(
                gba, h2_hbm.at[pl.ds(c * npad + base_row + roff, CH)])

        plsc.subcore_barrier()

        coff_c = c * npad

        def proc(j, buf, sem):
            pltpu.make_async_copy(h2_hbm.at[idx_r.at[j]], buf, sem).wait()

            @pl.loop(0, CH // L)
            def _grp(g):
                goff = pl.multiple_of(g * L, L)
                ev = ewb[j, pl.ds(goff, L)]
                for i in range(L):
                    w = ev[i]
                    e_idx = goff + i
                    for q4 in range(qn // 2):
                        v = buf[e_idx, pl.ds(q4 * L, L)]
                        lo = lax.bitcast_convert_type(v << 16, jnp.float32)
                        hi = lax.bitcast_convert_type(
                            v & jnp.int32(-65536), jnp.float32)
                        rows_a[e_idx, pl.ds(q4 * 2 * L, L)] = lo * w
                        rows_a[e_idx, pl.ds((q4 * 2 + 1) * L, L)] = hi * w

            pltpu.sync_copy(rows_a, acc_sh.at[idx_c.at[j]], add=True)

        def issue(j, buf, sem):
            pltpu.async_copy(h2_hbm.at[idx_r.at[j]], buf, sem)

        for b in range(nblk):
            pltpu.sync_copy(row3d.at[wid, b], idx_r)
            pltpu.sync_copy(col3d.at[wid, b], idx_c)
            pltpu.sync_copy(ew3d.at[wid, b], ewb)

            @pl.loop(0, bs)
            def _rb(j):
                for g in range(CH // L):
                    idx_r[j, pl.ds(g * L, L)] = (
                        idx_r[j, pl.ds(g * L, L)] + coff_c)

            issue(0, gba, sga)
            issue(1, gbb, sgb)

            @pl.loop(0, bs // 2)
            def _pair(it):
                j0 = 2 * it
                proc(j0, gba, sga)

                @pl.when(j0 + 2 < bs)
                def _():
                    issue(j0 + 2, gba, sga)

                proc(j0 + 1, gbb, sgb)

                @pl.when(j0 + 3 < bs)
                def _():
                    issue(j0 + 3, gbb, sgb)

            if bs % 2 == 1:
                proc(bs - 1, gba, sga)

        plsc.subcore_barrier()

        pltpu.sync_copy(acc_sh.at[pl.ds(base_row, rpt)],
                        out_hbm.at[c, pl.ds(base_row, rpt)])

    mesh = plsc.VectorSubcoreMesh(core_axis_name="c", subcore_axis_name="s")
    return pl.kernel(
        body,
        out_type=(
            jax.ShapeDtypeStruct((NC, npad, d_out), jnp.float32),
            jax.ShapeDtypeStruct((NC * npad, d_out // 2), jnp.int32),
            jax.ShapeDtypeStruct((NC, npad), jnp.float32),
        ),
        mesh=mesh,
        compiler_params=pltpu.CompilerParams(needs_layout_passes=False,
                                             use_tc_tiling_on_sc=False),
        scratch_types=[
            pltpu.VMEM((bs, CH), jnp.int32),
            pltpu.VMEM((bs, CH), jnp.int32),
            pltpu.VMEM((bs, CH), jnp.float32),
            pltpu.VMEM((CH, d_out), jnp.float32),
            pltpu.VMEM((CH, d_out // 2), jnp.int32),
            pltpu.VMEM((CH, d_out // 2), jnp.int32),
            pltpu.VMEM((8, d_out), jnp.float32),
            pltpu.VMEM((rpt,), jnp.float32),
            pltpu.VMEM_SHARED((npad, d_out), jnp.float32),
            pltpu.VMEM_SHARED((npad,), jnp.float32),
            pltpu.SemaphoreType.DMA,
            pltpu.SemaphoreType.DMA,
        ],
    )


def _matmul_body(x_ref, w_ref, o_ref):
    o_ref[...] = jnp.dot(x_ref[...], w_ref[...],
                         preferred_element_type=jnp.float32)


def _combine_body(p_ref, d_ref, o_ref):
    o_ref[...] = (p_ref[0] + p_ref[1]) * d_ref[0][:, None]


def kernel(x, edge_index, edge_weight, W):
    n, d_in = x.shape
    d_out = W.shape[1]
    e = edge_weight.shape[0]

    loop_idx = jnp.arange(n, dtype=edge_index.dtype)
    row = jnp.concatenate([edge_index[0], loop_idx])
    col = jnp.concatenate([edge_index[1], loop_idx])
    ew = jnp.concatenate([edge_weight, jnp.ones((n,), edge_weight.dtype)])
    e_tot = e + n
    grp = NW * CH
    e_pad = ((e_tot + grp - 1) // grp) * grp
    pad = e_pad - e_tot
    cpw = e_pad // CH // NW
    nblk = 3 if cpw % 3 == 0 else 1
    shp = (NW, nblk, cpw // nblk, CH)
    row = jnp.concatenate([row, jnp.zeros((pad,), row.dtype)]).reshape(shp)
    col = jnp.concatenate([col, jnp.zeros((pad,), col.dtype)]).reshape(shp)
    ew = jnp.concatenate([ew, jnp.zeros((pad,), ew.dtype)]).reshape(shp)

    rpt = ((n + NS * CH - 1) // (NS * CH)) * CH
    npad = rpt * NS

    xp = jnp.concatenate(
        [x, jnp.zeros((npad - n, d_in), x.dtype)]) if npad > n else x
    bm = 1024
    h = pl.pallas_call(
        _matmul_body,
        grid=(npad // bm,),
        in_specs=[pl.BlockSpec((bm, d_in), lambda i: (i, 0)),
                  pl.BlockSpec((d_in, d_out), lambda i: (0, 0))],
        out_specs=pl.BlockSpec((bm, d_out), lambda i: (i, 0)),
        out_shape=jax.ShapeDtypeStruct((npad, d_out), jnp.float32),
    )(xp, W)

    partial, _h2, dis = _make_sc_kernel(npad, d_out, nblk, cpw // nblk)(
        row, col, ew, h)

    out = pl.pallas_call(
        _combine_body,
        grid=(npad // bm,),
        in_specs=[pl.BlockSpec((NC, bm, d_out), lambda i: (0, i, 0)),
                  pl.BlockSpec((NC, bm), lambda i: (0, i))],
        out_specs=pl.BlockSpec((bm, d_out), lambda i: (i, 0)),
        out_shape=jax.ShapeDtypeStruct((npad, d_out), jnp.float32),
    )(partial, dis)
    return out[:n]
